# Initial kernel scaffold; baseline (speedup 1.0000x reference)
import jax
import jax.numpy as jnp
from jax.experimental import pallas as pl


def kernel(img):
    raise NotImplementedError("write your pallas kernel here")


# speedup vs baseline: 74.9290x; 74.9290x over previous
import functools

import jax
import jax.numpy as jnp
from jax import lax
from jax.experimental import pallas as pl
from jax.experimental.pallas import tpu as pltpu
from jax.experimental.pallas import tpu_sc as plsc

N = 16777216
NBIN = 256
INV_DH = float(NBIN - 1)
UMAX = float(NBIN - 1)

NC = 2
NS = 16
NW = NC * NS
LANES = 16

PER_W = N // NW
CH = 32768
NCH = PER_W // CH
VEC_PER_CH = CH // LANES
UNROLL = 8

STRIDE = 272
HIST_WORDS = LANES * STRIDE

@functools.cache
def _build_hist_kernel():
    mesh = plsc.VectorSubcoreMesh(core_axis_name="c", subcore_axis_name="s")
    return pl.kernel(
        _hist_body,
        out_type=jax.ShapeDtypeStruct((NW, STRIDE), jnp.float32),
        mesh=mesh,
        compiler_params=pltpu.CompilerParams(needs_layout_passes=False),
        scratch_types=[
            pltpu.VMEM((CH,), jnp.float32),
            pltpu.VMEM((CH,), jnp.float32),
            pltpu.VMEM((HIST_WORDS,), jnp.float32),
            pltpu.VMEM((STRIDE,), jnp.float32),
            pltpu.SemaphoreType.DMA,
            pltpu.SemaphoreType.DMA,
        ],
    )


def _hist_body(img_hbm, out_hbm, buf0, buf1, hist, outrow, sem0, sem1):
    wid = lax.axis_index("s") * NC + lax.axis_index("c")
    base = wid * PER_W

    zero = jnp.zeros((LANES,), jnp.float32)

    def zbody(i, carry):
        hist[pl.ds(i * LANES, LANES)] = zero
        return carry

    lax.fori_loop(0, HIST_WORDS // LANES, zbody, 0)

    lane_base = lax.iota(jnp.int32, LANES) * STRIDE

    def process(bref):
        def pbody(i, carry):
            off = i * (UNROLL * LANES)
            for j in range(UNROLL):
                x = bref[pl.ds(off + j * LANES, LANES)]
                u = jnp.minimum(jnp.maximum(x * INV_DH, 0.0), UMAX)
                idx = u.astype(jnp.int32)
                d = u - idx.astype(jnp.float32)
                f1 = lane_base + idx
                plsc.addupdate_scatter(hist, [f1], 1.0 - d)
                plsc.addupdate_scatter(hist, [f1 + 1], d)
            return carry

        lax.fori_loop(0, VEC_PER_CH // UNROLL, pbody, 0)

    def copy(c, bref, sem):
        return pltpu.make_async_copy(
            img_hbm.at[pl.ds(base + c * CH, CH)], bref, sem
        )

    copy(0, buf0, sem0).start()

    def chunk_pair(p, carry):
        c0 = 2 * p
        copy(c0, buf0, sem0).wait()
        copy(c0 + 1, buf1, sem1).start()
        process(buf0)
        copy(c0 + 1, buf1, sem1).wait()

        @pl.when(p < NCH // 2 - 1)
        def _():
            copy(c0 + 2, buf0, sem0).start()

        process(buf1)
        return carry

    lax.fori_loop(0, NCH // 2, chunk_pair, 0)

    for v in range(STRIDE // LANES):
        acc = hist[pl.ds(v * LANES, LANES)]
        for l in range(1, LANES):
            acc = acc + hist[pl.ds(l * STRIDE + v * LANES, LANES)]
        outrow[pl.ds(v * LANES, LANES)] = acc

    pltpu.sync_copy(outrow, out_hbm.at[wid])


def kernel(img):
    img = img.reshape(-1)
    parts = _build_hist_kernel()(img)
    return jnp.sum(parts[:, :NBIN], axis=0)


# speedup vs baseline: 275.0898x; 3.6713x over previous
import functools

import jax
import jax.numpy as jnp
from jax import lax
from jax.experimental import pallas as pl
from jax.experimental.pallas import tpu as pltpu
from jax.experimental.pallas import tpu_sc as plsc

N = 16777216
NBIN = 256
INV_DH = float(NBIN - 1)
UMAX = float(NBIN - 1)

NC = 2
NS = 16
NW = NC * NS
LANES = 16

PER_W = N // NW
CH = 32768
NCH = PER_W // CH
VEC_PER_CH = CH // LANES
UNROLL = 8

STRIDE = 272
HIST_WORDS = LANES * STRIDE

@functools.cache
def _build_hist_kernel():
    mesh = plsc.VectorSubcoreMesh(core_axis_name="c", subcore_axis_name="s")
    return pl.kernel(
        _hist_body,
        out_type=jax.ShapeDtypeStruct((NW, STRIDE), jnp.float32),
        mesh=mesh,
        compiler_params=pltpu.CompilerParams(needs_layout_passes=False),
        scratch_types=[
            pltpu.VMEM((CH,), jnp.float32),
            pltpu.VMEM((CH,), jnp.float32),
            pltpu.VMEM((HIST_WORDS,), jnp.float32),
            pltpu.VMEM((STRIDE,), jnp.float32),
            pltpu.SemaphoreType.DMA,
            pltpu.SemaphoreType.DMA,
        ],
    )


def _hist_body(img_hbm, out_hbm, buf0, buf1, hist, outrow, sem0, sem1):
    wid = lax.axis_index("s") * NC + lax.axis_index("c")
    base = wid * PER_W

    zero = jnp.zeros((LANES,), jnp.float32)

    def zbody(i, carry):
        hist[pl.ds(i * LANES, LANES)] = zero
        return carry

    lax.fori_loop(0, HIST_WORDS // LANES, zbody, 0)

    lane_base = lax.iota(jnp.int32, LANES) * STRIDE

    def process(bref):
        @plsc.parallel_loop(0, CH, LANES, unroll=UNROLL)
        def _(i):
            x = bref[pl.ds(i, LANES)]
            u = jnp.minimum(jnp.maximum(x * INV_DH, 0.0), UMAX)
            idx = u.astype(jnp.int32)
            d = u - idx.astype(jnp.float32)
            f1 = lane_base + idx
            plsc.addupdate_scatter(hist, [f1], 1.0 - d)
            plsc.addupdate_scatter(hist, [f1 + 1], d)

    def copy(c, bref, sem):
        return pltpu.make_async_copy(
            img_hbm.at[pl.ds(base + c * CH, CH)], bref, sem
        )

    copy(0, buf0, sem0).start()

    def chunk_pair(p, carry):
        c0 = 2 * p
        copy(c0, buf0, sem0).wait()
        copy(c0 + 1, buf1, sem1).start()
        process(buf0)
        copy(c0 + 1, buf1, sem1).wait()

        @pl.when(p < NCH // 2 - 1)
        def _():
            copy(c0 + 2, buf0, sem0).start()

        process(buf1)
        return carry

    lax.fori_loop(0, NCH // 2, chunk_pair, 0)

    for v in range(STRIDE // LANES):
        acc = hist[pl.ds(v * LANES, LANES)]
        for l in range(1, LANES):
            acc = acc + hist[pl.ds(l * STRIDE + v * LANES, LANES)]
        outrow[pl.ds(v * LANES, LANES)] = acc

    pltpu.sync_copy(outrow, out_hbm.at[wid])


def kernel(img):
    img = img.reshape(-1)
    parts = _build_hist_kernel()(img)
    return jnp.sum(parts[:, :NBIN], axis=0)
